# baseline (speedup 1.0000x reference)
import functools

import jax
import jax.numpy as jnp
from jax import lax
from jax.experimental import pallas as pl
from jax.experimental.pallas import tpu as pltpu
from jax.experimental.pallas import tpu_sc as plsc

N = 10000
E = 320000
D_IN = 128
D_HID = 64
D_OUT = 128
R = 8
RN = R * N

NC = 2
NS = 16
NW = NC * NS
L = 16

EPT = E // NW
G = 80
STEPS = EPT // G
ROWS_PT = N // NS
ZROWS = 25

_mesh = plsc.VectorSubcoreMesh(core_axis_name="c", subcore_axis_name="s")
_sc_params = pltpu.CompilerParams(use_tc_tiling_on_sc=False,
                                  needs_layout_passes=False)


CH = 2000
GPC = CH // G
NCHK = EPT // CH
NBUF = 3
QPC = (GPC - 1) // NBUF
DEG_SLICE = 5008
RNP = NS * DEG_SLICE
EPS = E // NS


def _edge_body(d_feat, with_table, hall_hbm, src_hbm, dst_hbm, et_hbm,
               recip_hbm, acc_out, norm_out, srcc, dstc, etc_, sidx,
               normc, rows_bufs, zb, acc, dbuf, onesb, degacc,
               sg, ss, sn):
  c = lax.axis_index("c")
  s = lax.axis_index("s")
  wid = s * NC + c
  base = wid * EPT
  nchunk = d_feat // L

  z16 = jnp.zeros((L,), jnp.float32)

  def zfill(i, _):
    for c4 in range(nchunk):
      zb[i, pl.ds(c4 * L, L)] = z16
    return 0

  lax.fori_loop(0, ZROWS, zfill, 0)

  for i in range(ROWS_PT // ZROWS):
    pltpu.sync_copy(zb, acc.at[pl.ds(s * ROWS_PT + i * ZROWS, ZROWS), :])

  if with_table:
    def dzfill(i, _):
      dbuf[pl.ds(i * L, L)] = z16
      return 0

    lax.fori_loop(0, DEG_SLICE // L, dzfill, 0)
    ones16 = jnp.ones((L,), jnp.float32)
    for k in range(G // L):
      onesb[pl.ds(k * L, L)] = ones16
    dslice = pl.ds(s * DEG_SLICE, DEG_SLICE)
    pltpu.sync_copy(dbuf, degacc.at[dslice])
    plsc.subcore_barrier()

    dbase = s * EPS

    def degchunk(ci, _):
      coff = dbase + ci * CH
      pltpu.sync_copy(dst_hbm.at[pl.ds(coff, CH)], dstc)
      pltpu.sync_copy(et_hbm.at[pl.ds(coff, CH)], etc_)

      def didxf(g, _):
        for q in range(G // L):
          o = pl.ds(g * G + q * L, L)
          sidx[g, pl.ds(q * L, L)] = etc_[o] * N + dstc[o]
        return 0

      lax.fori_loop(0, GPC, didxf, 0)

      def dfire(g, _):
        pltpu.async_copy(onesb, degacc.at[sidx.at[g]], sn, add=True)
        return 0

      lax.fori_loop(0, GPC, dfire, 0)

      def ddrain(g, _):
        pltpu.make_async_copy(onesb, degacc.at[sidx.at[0]], sn).wait()
        return 0

      lax.fori_loop(0, GPC, ddrain, 0)
      return 0

    lax.fori_loop(0, EPS // CH, degchunk, 0)
    plsc.subcore_barrier()

    pltpu.sync_copy(degacc.at[dslice], dbuf)

    def recipf(i, _):
      v = dbuf[pl.ds(i * L, L)]
      dbuf[pl.ds(i * L, L)] = 1.0 / jnp.maximum(v, 1.0)
      return 0

    lax.fori_loop(0, DEG_SLICE // L, recipf, 0)
    pltpu.sync_copy(dbuf, degacc.at[dslice])
  plsc.subcore_barrier()

  def fire_g(g, rows, sem):
    pltpu.async_copy(hall_hbm.at[srcc.at[pl.ds(g * G, G)]], rows, sem)

  def wait_g(rows, sem):
    pltpu.make_async_copy(hall_hbm.at[srcc.at[pl.ds(0, G)]], rows, sem).wait()

  def fire_s(g, rows, sem):
    pltpu.async_copy(rows, acc.at[sidx.at[g]], sem, add=True)

  def wait_s(rows, sem):
    pltpu.make_async_copy(rows, acc.at[sidx.at[0]], sem).wait()

  def scale(rows, goff):
    def rowscale(r, _):
      for u in range(2):
        sp = plsc.load_gather(normc, [jnp.full((L,), goff + 2 * r + u,
                                               jnp.int32)])
        for c4 in range(nchunk):
          rows[2 * r + u, pl.ds(c4 * L, L)] = (
              rows[2 * r + u, pl.ds(c4 * L, L)] * sp)
      return 0

    lax.fori_loop(0, G // 2, rowscale, 0)

  def chunk(ci, _):
    coff = base + ci * CH
    pltpu.sync_copy(src_hbm.at[pl.ds(coff, CH)], srcc)
    pltpu.sync_copy(dst_hbm.at[pl.ds(coff, CH)], dstc)
    pltpu.sync_copy(et_hbm.at[pl.ds(coff, CH)], etc_)

    def cidx(g, _):
      for q in range(G // L):
        o = pl.ds(g * G + q * L, L)
        s16 = srcc[o]
        d16 = dstc[o]
        t16 = etc_[o]
        srcc[o] = t16 * N + s16
        sidx[g, pl.ds(q * L, L)] = d16
        if with_table:
          etc_[o] = t16 * N + d16
      return 0

    lax.fori_loop(0, GPC, cidx, 0)

    if with_table:
      def nfire(g, _):
        pltpu.async_copy(degacc.at[etc_.at[pl.ds(g * G, G)]],
                         normc.at[pl.ds(g * G, G)], sn)
        return 0

      lax.fori_loop(0, GPC, nfire, 0)
    else:
      pltpu.sync_copy(recip_hbm.at[pl.ds(coff, CH)], normc)

    for j in range(NBUF - 1):
      fire_g(j, rows_bufs[j], sg[j])

    if with_table:
      def ndrain(g, _):
        pltpu.make_async_copy(degacc.at[etc_.at[pl.ds(0, G)]],
                              normc.at[pl.ds(0, G)], sn).wait()
        return 0

      lax.fori_loop(0, GPC, ndrain, 0)
      pltpu.sync_copy(normc, norm_out.at[pl.ds(coff, CH)])

    def quad(i, _):
      for j in range(NBUF):
        g = NBUF * i + j
        wait_g(rows_bufs[j], sg[j])
        scale(rows_bufs[j], g * G)
        fire_s(g, rows_bufs[j], ss[j])
        jn = (j + NBUF - 1) % NBUF
        if j == 0:
          @pl.when(i > 0)
          def _():
            wait_s(rows_bufs[jn], ss[jn])

          fire_g(g + NBUF - 1, rows_bufs[jn], sg[jn])
        elif NBUF * (QPC - 1) + j + NBUF - 1 <= GPC - 1:
          wait_s(rows_bufs[jn], ss[jn])
          fire_g(g + NBUF - 1, rows_bufs[jn], sg[jn])
        else:
          @pl.when(i < QPC - 1)
          def _():
            wait_s(rows_bufs[jn], ss[jn])
            fire_g(g + NBUF - 1, rows_bufs[jn], sg[jn])

      return 0

    lax.fori_loop(0, QPC, quad, 0)

    wait_g(rows_bufs[0], sg[0])
    scale(rows_bufs[0], (GPC - 1) * G)
    fire_s(GPC - 1, rows_bufs[0], ss[0])
    for j in range(NBUF):
      wait_s(rows_bufs[j], ss[j])
    return 0

  lax.fori_loop(0, NCHK, chunk, 0)
  plsc.subcore_barrier()
  for i in range(ROWS_PT // ZROWS):
    pltpu.sync_copy(acc.at[pl.ds(s * ROWS_PT + i * ZROWS, ZROWS), :], zb)
    pltpu.sync_copy(zb, acc_out.at[c, pl.ds(s * ROWS_PT + i * ZROWS, ZROWS), :])


def _edge_call(d_feat, with_table):
  scratch = [
      pltpu.VMEM((CH,), jnp.int32),
      pltpu.VMEM((CH,), jnp.int32),
      pltpu.VMEM((CH,), jnp.int32),
      pltpu.VMEM((GPC, G), jnp.int32),
      pltpu.VMEM((CH,), jnp.float32),
  ]
  scratch += [pltpu.VMEM((G, d_feat), jnp.float32) for _ in range(NBUF)]
  scratch += [
      pltpu.VMEM((ZROWS, d_feat), jnp.float32),
      pltpu.VMEM_SHARED((N, d_feat), jnp.float32),
  ]
  if with_table:
    scratch.append(pltpu.VMEM((DEG_SLICE,), jnp.float32))
    scratch.append(pltpu.VMEM((G,), jnp.float32))
    scratch.append(pltpu.VMEM_SHARED((RNP,), jnp.float32))
  scratch += [pltpu.SemaphoreType.DMA] * (2 * NBUF + 1)
  out_type = [jax.ShapeDtypeStruct((NC, N, d_feat), jnp.float32)]
  if with_table:
    out_type.append(jax.ShapeDtypeStruct((E,), jnp.float32))

  def body(*args):
    if with_table:
      (hall, src, dst, et, acc_out, norm_out,
       srcc, dstc, etc_, sidx, normc, *rest) = args
      recip = None
    else:
      (hall, src, dst, et, recip, acc_out,
       srcc, dstc, etc_, sidx, normc, *rest) = args
      norm_out = None
    rows_bufs = list(rest[:NBUF])
    rest = rest[NBUF:]
    if with_table:
      zb, acc, dbuf, onesb, degacc, *sems = rest
    else:
      zb, acc, *sems = rest
      dbuf = onesb = degacc = None
    sg = list(sems[:NBUF])
    ss = list(sems[NBUF:2 * NBUF])
    sn = sems[2 * NBUF]
    _edge_body(d_feat, with_table, hall, src, dst, et, recip, acc_out,
               norm_out, srcc, dstc, etc_, sidx, normc,
               rows_bufs, zb, acc, dbuf, onesb, degacc, sg, ss, sn)

  return pl.kernel(body, out_type=out_type, mesh=_mesh,
                   scratch_types=scratch, compiler_params=_sc_params)


_edge1_call = _edge_call(D_HID, True)
_edge2_call = _edge_call(D_OUT, False)


NB = 10
BN = N // NB


def _tc1_body(x_ref, w_ref, root_ref, b_ref, hall_ref, xr_ref):
  xb = x_ref[...]
  hall_ref[...] = jnp.dot(xb, w_ref[0], preferred_element_type=jnp.float32)

  @pl.when(pl.program_id(1) == 0)
  def _():
    xr_ref[...] = (jnp.dot(xb, root_ref[...],
                           preferred_element_type=jnp.float32) + b_ref[...])


def _tc1(x, W1, root1, b1):
  return pl.pallas_call(
      _tc1_body,
      grid=(NB, R),
      in_specs=[
          pl.BlockSpec((BN, D_IN), lambda i, r: (i, 0)),
          pl.BlockSpec((1, D_IN, D_HID), lambda i, r: (r, 0, 0)),
          pl.BlockSpec((D_IN, D_HID), lambda i, r: (0, 0)),
          pl.BlockSpec((1, D_HID), lambda i, r: (0, 0)),
      ],
      out_specs=[
          pl.BlockSpec((BN, D_HID), lambda i, r: (r * NB + i, 0)),
          pl.BlockSpec((BN, D_HID), lambda i, r: (i, 0)),
      ],
      out_shape=[
          jax.ShapeDtypeStruct((RN, D_HID), jnp.float32),
          jax.ShapeDtypeStruct((N, D_HID), jnp.float32),
      ],
  )(x, W1, root1, b1)


def _tc2_body(acc_ref, xr_ref, w_ref, root_ref, b_ref, hall_ref, xr2_ref):
  hb = jnp.maximum(acc_ref[0] + acc_ref[1] + xr_ref[...], 0.0)
  hall_ref[...] = jnp.dot(hb, w_ref[0], preferred_element_type=jnp.float32)

  @pl.when(pl.program_id(1) == 0)
  def _():
    xr2_ref[...] = (jnp.dot(hb, root_ref[...],
                            preferred_element_type=jnp.float32) + b_ref[...])


def _tc2(acc1, xr1, W2, root2, b2):
  return pl.pallas_call(
      _tc2_body,
      grid=(NB, R),
      in_specs=[
          pl.BlockSpec((2, BN, D_HID), lambda i, r: (0, i, 0)),
          pl.BlockSpec((BN, D_HID), lambda i, r: (i, 0)),
          pl.BlockSpec((1, D_HID, D_OUT), lambda i, r: (r, 0, 0)),
          pl.BlockSpec((D_HID, D_OUT), lambda i, r: (0, 0)),
          pl.BlockSpec((1, D_OUT), lambda i, r: (0, 0)),
      ],
      out_specs=[
          pl.BlockSpec((BN, D_OUT), lambda i, r: (r * NB + i, 0)),
          pl.BlockSpec((BN, D_OUT), lambda i, r: (i, 0)),
      ],
      out_shape=[
          jax.ShapeDtypeStruct((RN, D_OUT), jnp.float32),
          jax.ShapeDtypeStruct((N, D_OUT), jnp.float32),
      ],
  )(acc1, xr1, W2, root2, b2)


def _tc3_body(acc_ref, xr_ref, out_ref):
  out_ref[...] = acc_ref[0] + acc_ref[1] + xr_ref[...]


def _tc3(acc2, xr2):
  return pl.pallas_call(
      _tc3_body,
      grid=(NB,),
      in_specs=[
          pl.BlockSpec((2, BN, D_OUT), lambda i: (0, i, 0)),
          pl.BlockSpec((BN, D_OUT), lambda i: (i, 0)),
      ],
      out_specs=pl.BlockSpec((BN, D_OUT), lambda i: (i, 0)),
      out_shape=jax.ShapeDtypeStruct((N, D_OUT), jnp.float32),
  )(acc2, xr2)


@jax.jit
def kernel(x, edge_index, edge_type, W1, root1, b1, W2, root2, b2):
  src = edge_index[0]
  dst = edge_index[1]

  hall1, xr1 = _tc1(x, W1, root1, b1.reshape(1, D_HID))
  acc1, norm_e = _edge1_call(hall1, src, dst, edge_type)
  hall2, xr2 = _tc2(acc1, xr1, W2, root2, b2.reshape(1, D_OUT))
  (acc2,) = _edge2_call(hall2, src, dst, edge_type, norm_e)
  return _tc3(acc2, xr2)


# speedup vs baseline: 1.1906x; 1.1906x over previous
import functools

import jax
import jax.numpy as jnp
from jax import lax
from jax.experimental import pallas as pl
from jax.experimental.pallas import tpu as pltpu
from jax.experimental.pallas import tpu_sc as plsc

N = 10000
E = 320000
D_IN = 128
D_HID = 64
D_OUT = 128
R = 8
RN = R * N

NC = 2
NS = 16
NW = NC * NS
L = 16

EPT = E // NW
G = 80
STEPS = EPT // G
ROWS_PT = N // NS
ZROWS = 25

_mesh = plsc.VectorSubcoreMesh(core_axis_name="c", subcore_axis_name="s")
_sc_params = pltpu.CompilerParams(use_tc_tiling_on_sc=False,
                                  needs_layout_passes=False)


CH = 2000
GPC = CH // G
NCHK = EPT // CH
NBUF = 3
QPC = (GPC - 1) // NBUF
DEG_SLICE = 5008
RNP = NS * DEG_SLICE
EPS = E // NS


def _edge_body(d_feat, with_table, hall_hbm, src_hbm, dst_hbm, et_hbm,
               recip_hbm, acc_out, norm_out, srcc, dstc, etc_, sidx,
               normc, rows_bufs, zb, acc, dbuf, onesb, degacc,
               sg, ss, sn):
  c = lax.axis_index("c")
  s = lax.axis_index("s")
  wid = s * NC + c
  base = wid * EPT
  nchunk = d_feat // L

  z16 = jnp.zeros((L,), jnp.float32)

  def zfill(i, _):
    for c4 in range(nchunk):
      zb[i, pl.ds(c4 * L, L)] = z16
    return 0

  lax.fori_loop(0, ZROWS, zfill, 0)

  for i in range(ROWS_PT // ZROWS):
    pltpu.sync_copy(zb, acc.at[pl.ds(s * ROWS_PT + i * ZROWS, ZROWS), :])

  if with_table:
    def dzfill(i, _):
      dbuf[pl.ds(i * L, L)] = z16
      return 0

    lax.fori_loop(0, DEG_SLICE // L, dzfill, 0)
    ones16 = jnp.ones((L,), jnp.float32)
    for k in range(G // L):
      onesb[pl.ds(k * L, L)] = ones16
    dslice = pl.ds(s * DEG_SLICE, DEG_SLICE)
    pltpu.sync_copy(dbuf, degacc.at[dslice])
    plsc.subcore_barrier()

    dbase = s * EPS

    def degchunk(ci, _):
      coff = dbase + ci * CH
      pltpu.sync_copy(dst_hbm.at[pl.ds(coff, CH)], dstc)
      pltpu.sync_copy(et_hbm.at[pl.ds(coff, CH)], etc_)

      def didxf(g, _):
        for q in range(G // L):
          o = pl.ds(g * G + q * L, L)
          sidx[g, pl.ds(q * L, L)] = etc_[o] * N + dstc[o]
        return 0

      lax.fori_loop(0, GPC, didxf, 0)

      def dfire(g, _):
        pltpu.async_copy(onesb, degacc.at[sidx.at[g]], sn, add=True)
        return 0

      lax.fori_loop(0, GPC, dfire, 0)

      def ddrain(g, _):
        pltpu.make_async_copy(onesb, degacc.at[sidx.at[0]], sn).wait()
        return 0

      lax.fori_loop(0, GPC, ddrain, 0)
      return 0

    lax.fori_loop(0, EPS // CH, degchunk, 0)
    plsc.subcore_barrier()

    pltpu.sync_copy(degacc.at[dslice], dbuf)

    def recipf(i, _):
      v = dbuf[pl.ds(i * L, L)]
      dbuf[pl.ds(i * L, L)] = 1.0 / jnp.maximum(v, 1.0)
      return 0

    lax.fori_loop(0, DEG_SLICE // L, recipf, 0)
    pltpu.sync_copy(dbuf, degacc.at[dslice])
  plsc.subcore_barrier()

  def fire_g(g, rows, sem):
    pltpu.async_copy(hall_hbm.at[srcc.at[pl.ds(g * G, G)]], rows, sem)

  def wait_g(rows, sem):
    pltpu.make_async_copy(hall_hbm.at[srcc.at[pl.ds(0, G)]], rows, sem).wait()

  def fire_s(g, rows, sem):
    pltpu.async_copy(rows, acc.at[sidx.at[g]], sem, add=True)

  def wait_s(rows, sem):
    pltpu.make_async_copy(rows, acc.at[sidx.at[0]], sem).wait()

  def scale(rows, goff):
    def rowscale(r, _):
      for u in range(2):
        sp = plsc.load_gather(normc, [jnp.full((L,), goff + 2 * r + u,
                                               jnp.int32)])
        for c4 in range(nchunk):
          rows[2 * r + u, pl.ds(c4 * L, L)] = (
              rows[2 * r + u, pl.ds(c4 * L, L)] * sp)
      return 0

    lax.fori_loop(0, G // 2, rowscale, 0)

  def chunk(ci, _):
    coff = base + ci * CH
    pltpu.sync_copy(src_hbm.at[pl.ds(coff, CH)], srcc)
    pltpu.sync_copy(dst_hbm.at[pl.ds(coff, CH)], dstc)
    pltpu.sync_copy(et_hbm.at[pl.ds(coff, CH)], etc_)

    def cidx(g, _):
      for q in range(G // L):
        o = pl.ds(g * G + q * L, L)
        s16 = srcc[o]
        d16 = dstc[o]
        t16 = etc_[o]
        srcc[o] = t16 * N + s16
        sidx[g, pl.ds(q * L, L)] = d16
        if with_table:
          etc_[o] = t16 * N + d16
      return 0

    lax.fori_loop(0, GPC, cidx, 0)

    if with_table:
      def nfire(g, _):
        pltpu.async_copy(degacc.at[etc_.at[pl.ds(g * G, G)]],
                         normc.at[pl.ds(g * G, G)], sn)
        return 0

      lax.fori_loop(0, GPC, nfire, 0)
    else:
      pltpu.sync_copy(recip_hbm.at[pl.ds(coff, CH)], normc)

    for j in range(NBUF - 1):
      fire_g(j, rows_bufs[j], sg[j])

    if with_table:
      def ndrain(g, _):
        pltpu.make_async_copy(degacc.at[etc_.at[pl.ds(0, G)]],
                              normc.at[pl.ds(0, G)], sn).wait()
        return 0

      lax.fori_loop(0, GPC, ndrain, 0)
      pltpu.sync_copy(normc, norm_out.at[pl.ds(coff, CH)])

    def quad(i, _):
      for j in range(NBUF):
        g = NBUF * i + j
        wait_g(rows_bufs[j], sg[j])
        scale(rows_bufs[j], g * G)
        fire_s(g, rows_bufs[j], ss[j])
        jn = (j + NBUF - 1) % NBUF
        if j == 0:
          @pl.when(i > 0)
          def _():
            wait_s(rows_bufs[jn], ss[jn])

          fire_g(g + NBUF - 1, rows_bufs[jn], sg[jn])
        elif NBUF * (QPC - 1) + j + NBUF - 1 <= GPC - 1:
          wait_s(rows_bufs[jn], ss[jn])
          fire_g(g + NBUF - 1, rows_bufs[jn], sg[jn])
        else:
          @pl.when(i < QPC - 1)
          def _():
            wait_s(rows_bufs[jn], ss[jn])
            fire_g(g + NBUF - 1, rows_bufs[jn], sg[jn])

      return 0

    lax.fori_loop(0, QPC, quad, 0)

    wait_g(rows_bufs[0], sg[0])
    scale(rows_bufs[0], (GPC - 1) * G)
    fire_s(GPC - 1, rows_bufs[0], ss[0])
    for j in range(NBUF):
      wait_s(rows_bufs[j], ss[j])
    return 0

  lax.fori_loop(0, NCHK, chunk, 0)
  plsc.subcore_barrier()
  for i in range(ROWS_PT // ZROWS):
    pltpu.sync_copy(acc.at[pl.ds(s * ROWS_PT + i * ZROWS, ZROWS), :], zb)
    pltpu.sync_copy(zb, acc_out.at[c, pl.ds(s * ROWS_PT + i * ZROWS, ZROWS), :])


def _edge_call(d_feat, with_table):
  scratch = [
      pltpu.VMEM((CH,), jnp.int32),
      pltpu.VMEM((CH,), jnp.int32),
      pltpu.VMEM((CH,), jnp.int32),
      pltpu.VMEM((GPC, G), jnp.int32),
      pltpu.VMEM((CH,), jnp.float32),
  ]
  scratch += [pltpu.VMEM((G, d_feat), jnp.float32) for _ in range(NBUF)]
  scratch += [
      pltpu.VMEM((ZROWS, d_feat), jnp.float32),
      pltpu.VMEM_SHARED((N, d_feat), jnp.float32),
  ]
  if with_table:
    scratch.append(pltpu.VMEM((DEG_SLICE,), jnp.float32))
    scratch.append(pltpu.VMEM((G,), jnp.float32))
    scratch.append(pltpu.VMEM_SHARED((RNP,), jnp.float32))
  scratch += [pltpu.SemaphoreType.DMA] * (2 * NBUF + 1)
  out_type = [jax.ShapeDtypeStruct((NC, N, d_feat), jnp.float32)]
  if with_table:
    out_type.append(jax.ShapeDtypeStruct((E,), jnp.float32))

  def body(*args):
    if with_table:
      (hall, src, dst, et, acc_out, norm_out,
       srcc, dstc, etc_, sidx, normc, *rest) = args
      recip = None
    else:
      (hall, src, dst, et, recip, acc_out,
       srcc, dstc, etc_, sidx, normc, *rest) = args
      norm_out = None
    rows_bufs = list(rest[:NBUF])
    rest = rest[NBUF:]
    if with_table:
      zb, acc, dbuf, onesb, degacc, *sems = rest
    else:
      zb, acc, *sems = rest
      dbuf = onesb = degacc = None
    sg = list(sems[:NBUF])
    ss = list(sems[NBUF:2 * NBUF])
    sn = sems[2 * NBUF]
    _edge_body(d_feat, with_table, hall, src, dst, et, recip, acc_out,
               norm_out, srcc, dstc, etc_, sidx, normc,
               rows_bufs, zb, acc, dbuf, onesb, degacc, sg, ss, sn)

  return pl.kernel(body, out_type=out_type, mesh=_mesh,
                   scratch_types=scratch, compiler_params=_sc_params)


_edge1_call = _edge_call(D_HID, True)
_edge2_call = _edge_call(D_OUT, False)


NB = 10
BN = N // NB


def _tc1_body(x_ref, w_ref, root_ref, b_ref, hall_ref, xr_ref):
  xb = x_ref[...]
  for r in range(R):
    hall_ref[r] = jnp.dot(xb, w_ref[r], preferred_element_type=jnp.float32)
  xr_ref[...] = (jnp.dot(xb, root_ref[...], preferred_element_type=jnp.float32)
                 + b_ref[...])


def _tc1(x, W1, root1, b1):
  return pl.pallas_call(
      _tc1_body,
      grid=(NB,),
      in_specs=[
          pl.BlockSpec((BN, D_IN), lambda i: (i, 0)),
          pl.BlockSpec((R, D_IN, D_HID), lambda i: (0, 0, 0)),
          pl.BlockSpec((D_IN, D_HID), lambda i: (0, 0)),
          pl.BlockSpec((1, D_HID), lambda i: (0, 0)),
      ],
      out_specs=[
          pl.BlockSpec((R, BN, D_HID), lambda i: (0, i, 0)),
          pl.BlockSpec((BN, D_HID), lambda i: (i, 0)),
      ],
      out_shape=[
          jax.ShapeDtypeStruct((R, N, D_HID), jnp.float32),
          jax.ShapeDtypeStruct((N, D_HID), jnp.float32),
      ],
  )(x, W1, root1, b1)


def _tc2_body(acc_ref, xr_ref, w_ref, root_ref, b_ref, hall_ref, xr2_ref):
  hb = jnp.maximum(acc_ref[0] + acc_ref[1] + xr_ref[...], 0.0)
  for r in range(R):
    hall_ref[r] = jnp.dot(hb, w_ref[r], preferred_element_type=jnp.float32)
  xr2_ref[...] = (jnp.dot(hb, root_ref[...], preferred_element_type=jnp.float32)
                  + b_ref[...])


def _tc2(acc1, xr1, W2, root2, b2):
  return pl.pallas_call(
      _tc2_body,
      grid=(NB,),
      in_specs=[
          pl.BlockSpec((2, BN, D_HID), lambda i: (0, i, 0)),
          pl.BlockSpec((BN, D_HID), lambda i: (i, 0)),
          pl.BlockSpec((R, D_HID, D_OUT), lambda i: (0, 0, 0)),
          pl.BlockSpec((D_HID, D_OUT), lambda i: (0, 0)),
          pl.BlockSpec((1, D_OUT), lambda i: (0, 0)),
      ],
      out_specs=[
          pl.BlockSpec((R, BN, D_OUT), lambda i: (0, i, 0)),
          pl.BlockSpec((BN, D_OUT), lambda i: (i, 0)),
      ],
      out_shape=[
          jax.ShapeDtypeStruct((R, N, D_OUT), jnp.float32),
          jax.ShapeDtypeStruct((N, D_OUT), jnp.float32),
      ],
  )(acc1, xr1, W2, root2, b2)


def _tc3_body(acc_ref, xr_ref, out_ref):
  out_ref[...] = acc_ref[0] + acc_ref[1] + xr_ref[...]


def _tc3(acc2, xr2):
  return pl.pallas_call(
      _tc3_body,
      grid=(NB,),
      in_specs=[
          pl.BlockSpec((2, BN, D_OUT), lambda i: (0, i, 0)),
          pl.BlockSpec((BN, D_OUT), lambda i: (i, 0)),
      ],
      out_specs=pl.BlockSpec((BN, D_OUT), lambda i: (i, 0)),
      out_shape=jax.ShapeDtypeStruct((N, D_OUT), jnp.float32),
  )(acc2, xr2)


@jax.jit
def kernel(x, edge_index, edge_type, W1, root1, b1, W2, root2, b2):
  src = edge_index[0]
  dst = edge_index[1]

  hall1, xr1 = _tc1(x, W1, root1, b1.reshape(1, D_HID))
  acc1, norm_e = _edge1_call(hall1.reshape(RN, D_HID), src, dst, edge_type)
  hall2, xr2 = _tc2(acc1, xr1, W2, root2, b2.reshape(1, D_OUT))
  (acc2,) = _edge2_call(hall2.reshape(RN, D_OUT), src, dst, edge_type, norm_e)
  return _tc3(acc2, xr2)


# speedup vs baseline: 1.2568x; 1.0556x over previous
import functools

import jax
import jax.numpy as jnp
from jax import lax
from jax.experimental import pallas as pl
from jax.experimental.pallas import tpu as pltpu
from jax.experimental.pallas import tpu_sc as plsc

N = 10000
E = 320000
D_IN = 128
D_HID = 64
D_OUT = 128
R = 8
RN = R * N

NC = 2
NS = 16
NW = NC * NS
L = 16

EPT = E // NW
G = 80
STEPS = EPT // G
ROWS_PT = N // NS
ZROWS = 25

_mesh = plsc.VectorSubcoreMesh(core_axis_name="c", subcore_axis_name="s")
_sc_params = pltpu.CompilerParams(use_tc_tiling_on_sc=False,
                                  needs_layout_passes=False)


CH = 2000
GPC = CH // G
NCHK = EPT // CH
NBUF1 = 4
NBUF2 = 3
DEG_SLICE = 5008
RNP = NS * DEG_SLICE
EPS = E // NS


def _edge_body(d_feat, with_table, hall_hbm, ei_hbm, et_hbm,
               recip_hbm, acc_out, norm_out, srcc, dstc, etc_, sidx,
               normc, rows_bufs, zb, acc, dbuf, onesb, degacc,
               sg, ss, sn):
  NBUF = len(rows_bufs)
  QPC = (GPC - 1) // NBUF
  c = lax.axis_index("c")
  s = lax.axis_index("s")
  wid = s * NC + c
  base = wid * EPT
  nchunk = d_feat // L

  z16 = jnp.zeros((L,), jnp.float32)

  def zfill(i, _):
    for c4 in range(nchunk):
      zb[i, pl.ds(c4 * L, L)] = z16
    return 0

  lax.fori_loop(0, ZROWS, zfill, 0)

  for i in range(ROWS_PT // ZROWS):
    pltpu.sync_copy(zb, acc.at[pl.ds(s * ROWS_PT + i * ZROWS, ZROWS), :])

  if with_table:
    def dzfill(i, _):
      dbuf[pl.ds(i * L, L)] = z16
      return 0

    lax.fori_loop(0, DEG_SLICE // L, dzfill, 0)
    ones16 = jnp.ones((L,), jnp.float32)
    for k in range(G // L):
      onesb[pl.ds(k * L, L)] = ones16
    dslice = pl.ds(s * DEG_SLICE, DEG_SLICE)
    pltpu.sync_copy(dbuf, degacc.at[dslice])
    plsc.subcore_barrier()

    dbase = s * EPS

    def degchunk(ci, _):
      coff = dbase + ci * CH
      pltpu.sync_copy(ei_hbm.at[1, pl.ds(coff, CH)], dstc)
      pltpu.sync_copy(et_hbm.at[pl.ds(coff, CH)], etc_)

      def didxf(g, _):
        for q in range(G // L):
          o = pl.ds(g * G + q * L, L)
          sidx[g, pl.ds(q * L, L)] = etc_[o] * N + dstc[o]
        return 0

      lax.fori_loop(0, GPC, didxf, 0)

      def dfire(g, _):
        pltpu.async_copy(onesb, degacc.at[sidx.at[g]], sn, add=True)
        return 0

      lax.fori_loop(0, GPC, dfire, 0)

      def ddrain(g, _):
        pltpu.make_async_copy(onesb, degacc.at[sidx.at[0]], sn).wait()
        return 0

      lax.fori_loop(0, GPC, ddrain, 0)
      return 0

    lax.fori_loop(0, EPS // CH, degchunk, 0)
    plsc.subcore_barrier()

    pltpu.sync_copy(degacc.at[dslice], dbuf)

    def recipf(i, _):
      v = dbuf[pl.ds(i * L, L)]
      dbuf[pl.ds(i * L, L)] = 1.0 / jnp.maximum(v, 1.0)
      return 0

    lax.fori_loop(0, DEG_SLICE // L, recipf, 0)
    pltpu.sync_copy(dbuf, degacc.at[dslice])
  plsc.subcore_barrier()

  def fire_g(g, rows, sem):
    pltpu.async_copy(hall_hbm.at[srcc.at[pl.ds(g * G, G)]], rows, sem)

  def wait_g(rows, sem):
    pltpu.make_async_copy(hall_hbm.at[srcc.at[pl.ds(0, G)]], rows, sem).wait()

  def fire_s(g, rows, sem):
    pltpu.async_copy(rows, acc.at[sidx.at[g]], sem, add=True)

  def wait_s(rows, sem):
    pltpu.make_async_copy(rows, acc.at[sidx.at[0]], sem).wait()

  def scale(rows, goff):
    def rowscale(r, _):
      for u in range(2):
        sp = plsc.load_gather(normc, [jnp.full((L,), goff + 2 * r + u,
                                               jnp.int32)])
        for c4 in range(nchunk):
          rows[2 * r + u, pl.ds(c4 * L, L)] = (
              rows[2 * r + u, pl.ds(c4 * L, L)] * sp)
      return 0

    lax.fori_loop(0, G // 2, rowscale, 0)

  def chunk(ci, _):
    coff = base + ci * CH
    pltpu.sync_copy(ei_hbm.at[0, pl.ds(coff, CH)], srcc)
    pltpu.sync_copy(ei_hbm.at[1, pl.ds(coff, CH)], dstc)
    pltpu.sync_copy(et_hbm.at[pl.ds(coff, CH)], etc_)

    def cidx(g, _):
      for q in range(G // L):
        o = pl.ds(g * G + q * L, L)
        s16 = srcc[o]
        d16 = dstc[o]
        t16 = etc_[o]
        srcc[o] = t16 * N + s16
        sidx[g, pl.ds(q * L, L)] = d16
        if with_table:
          etc_[o] = t16 * N + d16
      return 0

    lax.fori_loop(0, GPC, cidx, 0)

    if with_table:
      def nfire(g, _):
        pltpu.async_copy(degacc.at[etc_.at[pl.ds(g * G, G)]],
                         normc.at[pl.ds(g * G, G)], sn)
        return 0

      lax.fori_loop(0, GPC, nfire, 0)
    else:
      pltpu.sync_copy(recip_hbm.at[pl.ds(coff, CH)], normc)

    for j in range(NBUF - 1):
      fire_g(j, rows_bufs[j], sg[j])

    if with_table:
      def ndrain(g, _):
        pltpu.make_async_copy(degacc.at[etc_.at[pl.ds(0, G)]],
                              normc.at[pl.ds(0, G)], sn).wait()
        return 0

      lax.fori_loop(0, GPC, ndrain, 0)
      pltpu.sync_copy(normc, norm_out.at[pl.ds(coff, CH)])

    def quad(i, _):
      for j in range(NBUF):
        g = NBUF * i + j
        wait_g(rows_bufs[j], sg[j])
        scale(rows_bufs[j], g * G)
        fire_s(g, rows_bufs[j], ss[j])
        jn = (j + NBUF - 1) % NBUF
        if j == 0:
          @pl.when(i > 0)
          def _():
            wait_s(rows_bufs[jn], ss[jn])

          fire_g(g + NBUF - 1, rows_bufs[jn], sg[jn])
        elif NBUF * (QPC - 1) + j + NBUF - 1 <= GPC - 1:
          wait_s(rows_bufs[jn], ss[jn])
          fire_g(g + NBUF - 1, rows_bufs[jn], sg[jn])
        else:
          @pl.when(i < QPC - 1)
          def _():
            wait_s(rows_bufs[jn], ss[jn])
            fire_g(g + NBUF - 1, rows_bufs[jn], sg[jn])

      return 0

    lax.fori_loop(0, QPC, quad, 0)

    wait_g(rows_bufs[0], sg[0])
    scale(rows_bufs[0], (GPC - 1) * G)
    fire_s(GPC - 1, rows_bufs[0], ss[0])
    for j in range(NBUF):
      wait_s(rows_bufs[j], ss[j])
    return 0

  lax.fori_loop(0, NCHK, chunk, 0)
  plsc.subcore_barrier()
  for i in range(ROWS_PT // ZROWS):
    pltpu.sync_copy(acc.at[pl.ds(s * ROWS_PT + i * ZROWS, ZROWS), :], zb)
    pltpu.sync_copy(zb, acc_out.at[c, pl.ds(s * ROWS_PT + i * ZROWS, ZROWS), :])


def _edge_call(d_feat, with_table, nbuf):
  scratch = [
      pltpu.VMEM((CH,), jnp.int32),
      pltpu.VMEM((CH,), jnp.int32),
      pltpu.VMEM((CH,), jnp.int32),
      pltpu.VMEM((GPC, G), jnp.int32),
      pltpu.VMEM((CH,), jnp.float32),
  ]
  scratch += [pltpu.VMEM((G, d_feat), jnp.float32) for _ in range(nbuf)]
  scratch += [
      pltpu.VMEM((ZROWS, d_feat), jnp.float32),
      pltpu.VMEM_SHARED((N, d_feat), jnp.float32),
  ]
  if with_table:
    scratch.append(pltpu.VMEM((DEG_SLICE,), jnp.float32))
    scratch.append(pltpu.VMEM((G,), jnp.float32))
    scratch.append(pltpu.VMEM_SHARED((RNP,), jnp.float32))
  scratch += [pltpu.SemaphoreType.DMA] * (2 * nbuf + 1)
  out_type = [jax.ShapeDtypeStruct((NC, N, d_feat), jnp.float32)]
  if with_table:
    out_type.append(jax.ShapeDtypeStruct((E,), jnp.float32))

  def body(*args):
    if with_table:
      (hall, ei, et, acc_out, norm_out,
       srcc, dstc, etc_, sidx, normc, *rest) = args
      recip = None
    else:
      (hall, ei, et, recip, acc_out,
       srcc, dstc, etc_, sidx, normc, *rest) = args
      norm_out = None
    rows_bufs = list(rest[:nbuf])
    rest = rest[nbuf:]
    if with_table:
      zb, acc, dbuf, onesb, degacc, *sems = rest
    else:
      zb, acc, *sems = rest
      dbuf = onesb = degacc = None
    sg = list(sems[:nbuf])
    ss = list(sems[nbuf:2 * nbuf])
    sn = sems[2 * nbuf]
    _edge_body(d_feat, with_table, hall, ei, et, recip, acc_out,
               norm_out, srcc, dstc, etc_, sidx, normc,
               rows_bufs, zb, acc, dbuf, onesb, degacc, sg, ss, sn)

  return pl.kernel(body, out_type=out_type, mesh=_mesh,
                   scratch_types=scratch, compiler_params=_sc_params)


_edge1_call = _edge_call(D_HID, True, NBUF1)
_edge2_call = _edge_call(D_OUT, False, NBUF2)


NB = 10
BN = N // NB


def _tc1_body(x_ref, w_ref, root_ref, b_ref, hall_ref, xr_ref):
  xb = x_ref[...]
  for r in range(R):
    hall_ref[r] = jnp.dot(xb, w_ref[r], preferred_element_type=jnp.float32)
  xr_ref[...] = (jnp.dot(xb, root_ref[...], preferred_element_type=jnp.float32)
                 + b_ref[...])


def _tc1(x, W1, root1, b1):
  return pl.pallas_call(
      _tc1_body,
      grid=(NB,),
      in_specs=[
          pl.BlockSpec((BN, D_IN), lambda i: (i, 0)),
          pl.BlockSpec((R, D_IN, D_HID), lambda i: (0, 0, 0)),
          pl.BlockSpec((D_IN, D_HID), lambda i: (0, 0)),
          pl.BlockSpec((1, D_HID), lambda i: (0, 0)),
      ],
      out_specs=[
          pl.BlockSpec((R, BN, D_HID), lambda i: (0, i, 0)),
          pl.BlockSpec((BN, D_HID), lambda i: (i, 0)),
      ],
      out_shape=[
          jax.ShapeDtypeStruct((R, N, D_HID), jnp.float32),
          jax.ShapeDtypeStruct((N, D_HID), jnp.float32),
      ],
  )(x, W1, root1, b1)


def _tc2_body(acc_ref, xr_ref, w_ref, root_ref, b_ref, hall_ref, xr2_ref):
  hb = jnp.maximum(acc_ref[0] + acc_ref[1] + xr_ref[...], 0.0)
  for r in range(R):
    hall_ref[r] = jnp.dot(hb, w_ref[r], preferred_element_type=jnp.float32)
  xr2_ref[...] = (jnp.dot(hb, root_ref[...], preferred_element_type=jnp.float32)
                  + b_ref[...])


def _tc2(acc1, xr1, W2, root2, b2):
  return pl.pallas_call(
      _tc2_body,
      grid=(NB,),
      in_specs=[
          pl.BlockSpec((2, BN, D_HID), lambda i: (0, i, 0)),
          pl.BlockSpec((BN, D_HID), lambda i: (i, 0)),
          pl.BlockSpec((R, D_HID, D_OUT), lambda i: (0, 0, 0)),
          pl.BlockSpec((D_HID, D_OUT), lambda i: (0, 0)),
          pl.BlockSpec((1, D_OUT), lambda i: (0, 0)),
      ],
      out_specs=[
          pl.BlockSpec((R, BN, D_OUT), lambda i: (0, i, 0)),
          pl.BlockSpec((BN, D_OUT), lambda i: (i, 0)),
      ],
      out_shape=[
          jax.ShapeDtypeStruct((R, N, D_OUT), jnp.float32),
          jax.ShapeDtypeStruct((N, D_OUT), jnp.float32),
      ],
  )(acc1, xr1, W2, root2, b2)


def _tc3_body(acc_ref, xr_ref, out_ref):
  out_ref[...] = acc_ref[0] + acc_ref[1] + xr_ref[...]


def _tc3(acc2, xr2):
  return pl.pallas_call(
      _tc3_body,
      grid=(NB,),
      in_specs=[
          pl.BlockSpec((2, BN, D_OUT), lambda i: (0, i, 0)),
          pl.BlockSpec((BN, D_OUT), lambda i: (i, 0)),
      ],
      out_specs=pl.BlockSpec((BN, D_OUT), lambda i: (i, 0)),
      out_shape=jax.ShapeDtypeStruct((N, D_OUT), jnp.float32),
  )(acc2, xr2)


@jax.jit
def kernel(x, edge_index, edge_type, W1, root1, b1, W2, root2, b2):
  hall1, xr1 = _tc1(x, W1, root1, b1.reshape(1, D_HID))
  acc1, norm_e = _edge1_call(hall1.reshape(RN, D_HID), edge_index, edge_type)
  hall2, xr2 = _tc2(acc1, xr1, W2, root2, b2.reshape(1, D_OUT))
  (acc2,) = _edge2_call(hall2.reshape(RN, D_OUT), edge_index, edge_type,
                        norm_e)
  return _tc3(acc2, xr2)


# speedup vs baseline: 1.3298x; 1.0581x over previous
import functools

import jax
import jax.numpy as jnp
from jax import lax
from jax.experimental import pallas as pl
from jax.experimental.pallas import tpu as pltpu
from jax.experimental.pallas import tpu_sc as plsc

N = 10000
E = 320000
D_IN = 128
D_HID = 64
D_OUT = 128
R = 8
RN = R * N

NC = 2
NS = 16
NW = NC * NS
L = 16

EPT = E // NW
G = 80
STEPS = EPT // G
ROWS_PT = N // NS
ZROWS = 25

_mesh = plsc.VectorSubcoreMesh(core_axis_name="c", subcore_axis_name="s")
_sc_params = pltpu.CompilerParams(use_tc_tiling_on_sc=False,
                                  needs_layout_passes=False)


CH = 2000
GPC = CH // G
NCHK = EPT // CH
NBUF1 = 4
NBUF2 = 3
DEG_SLICE = 5008
RNP = NS * DEG_SLICE
EPS = E // NS


def _edge_body(d_feat, with_table, hall_hbm, ei_hbm, et_hbm,
               recip_hbm, acc_out, norm_out, srcc, dstc, etc_, sidx,
               normc, rows_bufs, zb, acc, dbuf, onesb, degacc,
               sg, ss, sn):
  NBUF = len(rows_bufs)
  QPC = (GPC - 1) // NBUF
  c = lax.axis_index("c")
  s = lax.axis_index("s")
  wid = s * NC + c
  base = wid * EPT
  nchunk = d_feat // L

  z16 = jnp.zeros((L,), jnp.float32)

  def zfill(i, _):
    for c4 in range(nchunk):
      zb[i, pl.ds(c4 * L, L)] = z16
    return 0

  lax.fori_loop(0, ZROWS, zfill, 0)

  for i in range(ROWS_PT // ZROWS):
    pltpu.sync_copy(zb, acc.at[pl.ds(s * ROWS_PT + i * ZROWS, ZROWS), :])

  if with_table:
    def dzfill(i, _):
      dbuf[pl.ds(i * L, L)] = z16
      return 0

    lax.fori_loop(0, DEG_SLICE // L, dzfill, 0)
    ones16 = jnp.ones((L,), jnp.float32)
    for k in range(G // L):
      onesb[pl.ds(k * L, L)] = ones16
    dslice = pl.ds(s * DEG_SLICE, DEG_SLICE)
    pltpu.sync_copy(dbuf, degacc.at[dslice])
    plsc.subcore_barrier()

    dbase = s * EPS

    def degchunk(ci, _):
      coff = dbase + ci * CH
      pltpu.sync_copy(ei_hbm.at[1, pl.ds(coff, CH)], dstc)
      pltpu.sync_copy(et_hbm.at[pl.ds(coff, CH)], etc_)

      def didxf(g, _):
        for q in range(G // L):
          o = pl.ds(g * G + q * L, L)
          sidx[g, pl.ds(q * L, L)] = etc_[o] * N + dstc[o]
        return 0

      lax.fori_loop(0, GPC, didxf, 0)

      def dfire(g, _):
        pltpu.async_copy(onesb, degacc.at[sidx.at[g]], sn, add=True)
        return 0

      lax.fori_loop(0, GPC, dfire, 0)

      def ddrain(g, _):
        pltpu.make_async_copy(onesb, degacc.at[sidx.at[0]], sn).wait()
        return 0

      lax.fori_loop(0, GPC, ddrain, 0)
      return 0

    lax.fori_loop(0, EPS // CH, degchunk, 0)
    plsc.subcore_barrier()

    pltpu.sync_copy(degacc.at[dslice], dbuf)

    def recipf(i, _):
      v = dbuf[pl.ds(i * L, L)]
      dbuf[pl.ds(i * L, L)] = 1.0 / jnp.maximum(v, 1.0)
      return 0

    lax.fori_loop(0, DEG_SLICE // L, recipf, 0)
    pltpu.sync_copy(dbuf, degacc.at[dslice])
  plsc.subcore_barrier()

  def fire_g(g, rows, sem):
    pltpu.async_copy(hall_hbm.at[srcc.at[pl.ds(g * G, G)]], rows, sem)

  def wait_g(rows, sem):
    pltpu.make_async_copy(hall_hbm.at[srcc.at[pl.ds(0, G)]], rows, sem).wait()

  def fire_s(g, rows, sem):
    pltpu.async_copy(rows, acc.at[sidx.at[g]], sem, add=True)

  def wait_s(rows, sem):
    pltpu.make_async_copy(rows, acc.at[sidx.at[0]], sem).wait()

  def scale(rows, goff):
    def rowscale(r, _):
      for u in range(2):
        sp = plsc.load_gather(normc, [jnp.full((L,), goff + 2 * r + u,
                                               jnp.int32)])
        for c4 in range(nchunk):
          rows[2 * r + u, pl.ds(c4 * L, L)] = (
              rows[2 * r + u, pl.ds(c4 * L, L)] * sp)
      return 0

    lax.fori_loop(0, G // 2, rowscale, 0)

  def chunk(ci, _):
    coff = base + ci * CH
    pltpu.sync_copy(ei_hbm.at[0, pl.ds(coff, CH)], srcc)
    pltpu.sync_copy(ei_hbm.at[1, pl.ds(coff, CH)], dstc)
    pltpu.sync_copy(et_hbm.at[pl.ds(coff, CH)], etc_)

    def cidx(g, _):
      for q in range(G // L):
        o = pl.ds(g * G + q * L, L)
        s16 = srcc[o]
        d16 = dstc[o]
        t16 = etc_[o]
        srcc[o] = t16 * N + s16
        sidx[g, pl.ds(q * L, L)] = d16
        if with_table:
          etc_[o] = t16 * N + d16
      return 0

    lax.fori_loop(0, GPC, cidx, 0)

    if with_table:
      def nfire(g, _):
        pltpu.async_copy(degacc.at[etc_.at[pl.ds(g * G, G)]],
                         normc.at[pl.ds(g * G, G)], sn)
        return 0

      lax.fori_loop(0, GPC, nfire, 0)
    else:
      pltpu.sync_copy(recip_hbm.at[pl.ds(coff, CH)], normc)

    for j in range(NBUF - 1):
      fire_g(j, rows_bufs[j], sg[j])

    if with_table:
      def ndrain(g, _):
        pltpu.make_async_copy(degacc.at[etc_.at[pl.ds(0, G)]],
                              normc.at[pl.ds(0, G)], sn).wait()
        return 0

      lax.fori_loop(0, GPC, ndrain, 0)
      pltpu.sync_copy(normc, norm_out.at[pl.ds(coff, CH)])

    def quad(i, _):
      for j in range(NBUF):
        g = NBUF * i + j
        wait_g(rows_bufs[j], sg[j])
        scale(rows_bufs[j], g * G)
        fire_s(g, rows_bufs[j], ss[j])
        jn = (j + NBUF - 1) % NBUF
        if j == 0:
          @pl.when(i > 0)
          def _():
            wait_s(rows_bufs[jn], ss[jn])

          fire_g(g + NBUF - 1, rows_bufs[jn], sg[jn])
        elif NBUF * (QPC - 1) + j + NBUF - 1 <= GPC - 1:
          wait_s(rows_bufs[jn], ss[jn])
          fire_g(g + NBUF - 1, rows_bufs[jn], sg[jn])
        else:
          @pl.when(i < QPC - 1)
          def _():
            wait_s(rows_bufs[jn], ss[jn])
            fire_g(g + NBUF - 1, rows_bufs[jn], sg[jn])

      return 0

    lax.fori_loop(0, QPC, quad, 0)

    wait_g(rows_bufs[0], sg[0])
    scale(rows_bufs[0], (GPC - 1) * G)
    fire_s(GPC - 1, rows_bufs[0], ss[0])
    for j in range(NBUF):
      wait_s(rows_bufs[j], ss[j])
    return 0

  lax.fori_loop(0, NCHK, chunk, 0)
  plsc.subcore_barrier()
  for i in range(ROWS_PT // ZROWS):
    pltpu.sync_copy(acc.at[pl.ds(s * ROWS_PT + i * ZROWS, ZROWS), :], zb)
    pltpu.sync_copy(zb, acc_out.at[c, pl.ds(s * ROWS_PT + i * ZROWS, ZROWS), :])


def _edge_call(d_feat, with_table, nbuf):
  scratch = [
      pltpu.VMEM((CH,), jnp.int32),
      pltpu.VMEM((CH,), jnp.int32),
      pltpu.VMEM((CH,), jnp.int32),
      pltpu.VMEM((GPC, G), jnp.int32),
      pltpu.VMEM((CH,), jnp.float32),
  ]
  scratch += [pltpu.VMEM((G, d_feat), jnp.float32) for _ in range(nbuf)]
  scratch += [
      pltpu.VMEM((ZROWS, d_feat), jnp.float32),
      pltpu.VMEM_SHARED((N, d_feat), jnp.float32),
  ]
  if with_table:
    scratch.append(pltpu.VMEM((DEG_SLICE,), jnp.float32))
    scratch.append(pltpu.VMEM((G,), jnp.float32))
    scratch.append(pltpu.VMEM_SHARED((RNP,), jnp.float32))
  scratch += [pltpu.SemaphoreType.DMA] * (2 * nbuf + 1)
  out_type = [jax.ShapeDtypeStruct((NC, N, d_feat), jnp.float32)]
  if with_table:
    out_type.append(jax.ShapeDtypeStruct((E,), jnp.float32))

  def body(*args):
    if with_table:
      (hall, ei, et, acc_out, norm_out,
       srcc, dstc, etc_, sidx, normc, *rest) = args
      recip = None
    else:
      (hall, ei, et, recip, acc_out,
       srcc, dstc, etc_, sidx, normc, *rest) = args
      norm_out = None
    rows_bufs = list(rest[:nbuf])
    rest = rest[nbuf:]
    if with_table:
      zb, acc, dbuf, onesb, degacc, *sems = rest
    else:
      zb, acc, *sems = rest
      dbuf = onesb = degacc = None
    sg = list(sems[:nbuf])
    ss = list(sems[nbuf:2 * nbuf])
    sn = sems[2 * nbuf]
    _edge_body(d_feat, with_table, hall, ei, et, recip, acc_out,
               norm_out, srcc, dstc, etc_, sidx, normc,
               rows_bufs, zb, acc, dbuf, onesb, degacc, sg, ss, sn)

  return pl.kernel(body, out_type=out_type, mesh=_mesh,
                   scratch_types=scratch, compiler_params=_sc_params)


_edge1_call = _edge_call(D_HID, True, NBUF1)
_edge2_call = _edge_call(D_OUT, False, NBUF2)


NB = 10
BN = N // NB


def _tc1_body(x_ref, x2_ref, w_ref, root_ref, b_ref, hall_ref, xr_ref):
  x2b = x2_ref[...]
  for r in range(R):
    hall_ref[r] = jnp.dot(x2b, w_ref[r], preferred_element_type=jnp.float32)
  xb = x_ref[...]
  xr_ref[...] = (jnp.dot(xb, root_ref[...], preferred_element_type=jnp.float32)
                 + b_ref[...])


NB1 = 5
BN1 = N // NB1


def _tc1(x, W1, root1, b1):
  wp = jnp.zeros((R, 2 * D_IN, 2 * D_HID), jnp.float32)
  wp = wp.at[:, :D_IN, :D_HID].set(W1).at[:, D_IN:, D_HID:].set(W1)
  return pl.pallas_call(
      _tc1_body,
      grid=(NB1,),
      in_specs=[
          pl.BlockSpec((BN1, D_IN), lambda i: (i, 0)),
          pl.BlockSpec((BN1 // 2, 2 * D_IN), lambda i: (i, 0)),
          pl.BlockSpec((R, 2 * D_IN, 2 * D_HID), lambda i: (0, 0, 0)),
          pl.BlockSpec((D_IN, D_HID), lambda i: (0, 0)),
          pl.BlockSpec((1, D_HID), lambda i: (0, 0)),
      ],
      out_specs=[
          pl.BlockSpec((R, BN1 // 2, 2 * D_HID), lambda i: (0, i, 0)),
          pl.BlockSpec((BN1, D_HID), lambda i: (i, 0)),
      ],
      out_shape=[
          jax.ShapeDtypeStruct((R, N // 2, 2 * D_HID), jnp.float32),
          jax.ShapeDtypeStruct((N, D_HID), jnp.float32),
      ],
  )(x, x.reshape(N // 2, 2 * D_IN), wp, root1, b1)


def _tc2_body(acc_ref, xr_ref, w_ref, root_ref, b_ref, hall_ref, xr2_ref):
  hb = jnp.maximum(acc_ref[0] + acc_ref[1] + xr_ref[...], 0.0)
  for r in range(R):
    hall_ref[r] = jnp.dot(hb, w_ref[r], preferred_element_type=jnp.float32)
  xr2_ref[...] = (jnp.dot(hb, root_ref[...], preferred_element_type=jnp.float32)
                  + b_ref[...])


def _tc2(acc1, xr1, W2, root2, b2):
  return pl.pallas_call(
      _tc2_body,
      grid=(NB,),
      in_specs=[
          pl.BlockSpec((2, BN, D_HID), lambda i: (0, i, 0)),
          pl.BlockSpec((BN, D_HID), lambda i: (i, 0)),
          pl.BlockSpec((R, D_HID, D_OUT), lambda i: (0, 0, 0)),
          pl.BlockSpec((D_HID, D_OUT), lambda i: (0, 0)),
          pl.BlockSpec((1, D_OUT), lambda i: (0, 0)),
      ],
      out_specs=[
          pl.BlockSpec((R, BN, D_OUT), lambda i: (0, i, 0)),
          pl.BlockSpec((BN, D_OUT), lambda i: (i, 0)),
      ],
      out_shape=[
          jax.ShapeDtypeStruct((R, N, D_OUT), jnp.float32),
          jax.ShapeDtypeStruct((N, D_OUT), jnp.float32),
      ],
  )(acc1, xr1, W2, root2, b2)


def _tc3_body(acc_ref, xr_ref, out_ref):
  out_ref[...] = acc_ref[0] + acc_ref[1] + xr_ref[...]


def _tc3(acc2, xr2):
  return pl.pallas_call(
      _tc3_body,
      grid=(NB,),
      in_specs=[
          pl.BlockSpec((2, BN, D_OUT), lambda i: (0, i, 0)),
          pl.BlockSpec((BN, D_OUT), lambda i: (i, 0)),
      ],
      out_specs=pl.BlockSpec((BN, D_OUT), lambda i: (i, 0)),
      out_shape=jax.ShapeDtypeStruct((N, D_OUT), jnp.float32),
  )(acc2, xr2)


@jax.jit
def kernel(x, edge_index, edge_type, W1, root1, b1, W2, root2, b2):
  hall1, xr1 = _tc1(x, W1, root1, b1.reshape(1, D_HID))
  acc1, norm_e = _edge1_call(hall1.reshape(RN, D_HID), edge_index, edge_type)
  hall2, xr2 = _tc2(acc1, xr1, W2, root2, b2.reshape(1, D_OUT))
  (acc2,) = _edge2_call(hall2.reshape(RN, D_OUT), edge_index, edge_type,
                        norm_e)
  return _tc3(acc2, xr2)


# speedup vs baseline: 1.3313x; 1.0011x over previous
import jax
import jax.numpy as jnp
from jax import lax
from jax.experimental import pallas as pl
from jax.experimental.pallas import tpu as pltpu
from jax.experimental.pallas import tpu_sc as plsc

N = 10000
E = 320000
D_IN = 128
D_HID = 64
D_OUT = 128
R = 8
RN = R * N

NC = 2
NS = 16
NW = NC * NS
L = 16

EPT = E // NW
G = 80
ROWS_PT = N // NS
ZROWS = 25

_mesh = plsc.VectorSubcoreMesh(core_axis_name="c", subcore_axis_name="s")
_sc_params = pltpu.CompilerParams(use_tc_tiling_on_sc=False,
                                  needs_layout_passes=False)


CH = 2000
GPC = CH // G
NCHK = EPT // CH
NBUF1 = 4
NBUF2 = 3
DEG_SLICE = 5008
RNP = NS * DEG_SLICE
EPS = E // NS


def _edge_body(d_feat, with_table, hall_hbm, ei_hbm, et_hbm,
               recip_hbm, acc_out, norm_out, srcc, dstc, etc_, sidx,
               normc, rows_bufs, zb, acc, dbuf, onesb, degacc,
               sg, ss, sn):
  NBUF = len(rows_bufs)
  QPC = (GPC - 1) // NBUF
  c = lax.axis_index("c")
  s = lax.axis_index("s")
  wid = s * NC + c
  base = wid * EPT
  nchunk = d_feat // L

  z16 = jnp.zeros((L,), jnp.float32)

  def zfill(i, _):
    for c4 in range(nchunk):
      zb[i, pl.ds(c4 * L, L)] = z16
    return 0

  lax.fori_loop(0, ZROWS, zfill, 0)

  for i in range(ROWS_PT // ZROWS):
    pltpu.sync_copy(zb, acc.at[pl.ds(s * ROWS_PT + i * ZROWS, ZROWS), :])

  if with_table:
    def dzfill(i, _):
      dbuf[pl.ds(i * L, L)] = z16
      return 0

    lax.fori_loop(0, DEG_SLICE // L, dzfill, 0)
    ones16 = jnp.ones((L,), jnp.float32)
    for k in range(G // L):
      onesb[pl.ds(k * L, L)] = ones16
    dslice = pl.ds(s * DEG_SLICE, DEG_SLICE)
    pltpu.sync_copy(dbuf, degacc.at[dslice])
    plsc.subcore_barrier()

    dbase = s * EPS

    def degchunk(ci, _):
      coff = dbase + ci * CH
      pltpu.sync_copy(ei_hbm.at[1, pl.ds(coff, CH)], dstc)
      pltpu.sync_copy(et_hbm.at[pl.ds(coff, CH)], etc_)

      def didxf(g, _):
        for q in range(G // L):
          o = pl.ds(g * G + q * L, L)
          sidx[g, pl.ds(q * L, L)] = etc_[o] * N + dstc[o]
        return 0

      lax.fori_loop(0, GPC, didxf, 0)

      def dfire(g, _):
        pltpu.async_copy(onesb, degacc.at[sidx.at[g]], sn, add=True)
        return 0

      lax.fori_loop(0, GPC, dfire, 0)

      def ddrain(g, _):
        pltpu.make_async_copy(onesb, degacc.at[sidx.at[0]], sn).wait()
        return 0

      lax.fori_loop(0, GPC, ddrain, 0)
      return 0

    lax.fori_loop(0, EPS // CH, degchunk, 0)
    plsc.subcore_barrier()

    pltpu.sync_copy(degacc.at[dslice], dbuf)

    def recipf(i, _):
      v = dbuf[pl.ds(i * L, L)]
      dbuf[pl.ds(i * L, L)] = 1.0 / jnp.maximum(v, 1.0)
      return 0

    lax.fori_loop(0, DEG_SLICE // L, recipf, 0)
    pltpu.sync_copy(dbuf, degacc.at[dslice])
  plsc.subcore_barrier()

  def fire_g(g, rows, sem):
    pltpu.async_copy(hall_hbm.at[srcc.at[pl.ds(g * G, G)]], rows, sem)

  def wait_g(rows, sem):
    pltpu.make_async_copy(hall_hbm.at[srcc.at[pl.ds(0, G)]], rows, sem).wait()

  def fire_s(g, rows, sem):
    pltpu.async_copy(rows, acc.at[sidx.at[g]], sem, add=True)

  def wait_s(rows, sem):
    pltpu.make_async_copy(rows, acc.at[sidx.at[0]], sem).wait()

  def scale(rows, goff):
    def rowscale(r, _):
      for u in range(2):
        sp = plsc.load_gather(normc, [jnp.full((L,), goff + 2 * r + u,
                                               jnp.int32)])
        for c4 in range(nchunk):
          rows[2 * r + u, pl.ds(c4 * L, L)] = (
              rows[2 * r + u, pl.ds(c4 * L, L)] * sp)
      return 0

    lax.fori_loop(0, G // 2, rowscale, 0)

  def chunk(ci, _):
    coff = base + ci * CH
    pltpu.sync_copy(ei_hbm.at[0, pl.ds(coff, CH)], srcc)
    pltpu.sync_copy(ei_hbm.at[1, pl.ds(coff, CH)], dstc)
    pltpu.sync_copy(et_hbm.at[pl.ds(coff, CH)], etc_)

    def cidx(g, _):
      for q in range(G // L):
        o = pl.ds(g * G + q * L, L)
        s16 = srcc[o]
        d16 = dstc[o]
        t16 = etc_[o]
        srcc[o] = t16 * N + s16
        sidx[g, pl.ds(q * L, L)] = d16
        if with_table:
          etc_[o] = t16 * N + d16
      return 0

    lax.fori_loop(0, GPC, cidx, 0)

    if with_table:
      def nfire(g, _):
        pltpu.async_copy(degacc.at[etc_.at[pl.ds(g * G, G)]],
                         normc.at[pl.ds(g * G, G)], sn)
        return 0

      lax.fori_loop(0, GPC, nfire, 0)
    else:
      pltpu.sync_copy(recip_hbm.at[pl.ds(coff, CH)], normc)

    for j in range(NBUF - 1):
      fire_g(j, rows_bufs[j], sg[j])

    if with_table:
      def ndrain(g, _):
        pltpu.make_async_copy(degacc.at[etc_.at[pl.ds(0, G)]],
                              normc.at[pl.ds(0, G)], sn).wait()
        return 0

      lax.fori_loop(0, GPC, ndrain, 0)
      pltpu.sync_copy(normc, norm_out.at[pl.ds(coff, CH)])

    def quad(i, _):
      for j in range(NBUF):
        g = NBUF * i + j
        wait_g(rows_bufs[j], sg[j])
        scale(rows_bufs[j], g * G)
        fire_s(g, rows_bufs[j], ss[j])
        jn = (j + NBUF - 1) % NBUF
        if j == 0:
          @pl.when(i > 0)
          def _():
            wait_s(rows_bufs[jn], ss[jn])

          fire_g(g + NBUF - 1, rows_bufs[jn], sg[jn])
        elif NBUF * (QPC - 1) + j + NBUF - 1 <= GPC - 1:
          wait_s(rows_bufs[jn], ss[jn])
          fire_g(g + NBUF - 1, rows_bufs[jn], sg[jn])
        else:
          @pl.when(i < QPC - 1)
          def _():
            wait_s(rows_bufs[jn], ss[jn])
            fire_g(g + NBUF - 1, rows_bufs[jn], sg[jn])

      return 0

    lax.fori_loop(0, QPC, quad, 0)

    wait_g(rows_bufs[0], sg[0])
    scale(rows_bufs[0], (GPC - 1) * G)
    fire_s(GPC - 1, rows_bufs[0], ss[0])
    for j in range(NBUF):
      wait_s(rows_bufs[j], ss[j])
    return 0

  lax.fori_loop(0, NCHK, chunk, 0)
  plsc.subcore_barrier()
  for i in range(ROWS_PT // ZROWS):
    pltpu.sync_copy(acc.at[pl.ds(s * ROWS_PT + i * ZROWS, ZROWS), :], zb)
    pltpu.sync_copy(zb, acc_out.at[c, pl.ds(s * ROWS_PT + i * ZROWS, ZROWS), :])


def _edge_call(d_feat, with_table, nbuf):
  scratch = [
      pltpu.VMEM((CH,), jnp.int32),
      pltpu.VMEM((CH,), jnp.int32),
      pltpu.VMEM((CH,), jnp.int32),
      pltpu.VMEM((GPC, G), jnp.int32),
      pltpu.VMEM((CH,), jnp.float32),
  ]
  scratch += [pltpu.VMEM((G, d_feat), jnp.float32) for _ in range(nbuf)]
  scratch += [
      pltpu.VMEM((ZROWS, d_feat), jnp.float32),
      pltpu.VMEM_SHARED((N, d_feat), jnp.float32),
  ]
  if with_table:
    scratch.append(pltpu.VMEM((DEG_SLICE,), jnp.float32))
    scratch.append(pltpu.VMEM((G,), jnp.float32))
    scratch.append(pltpu.VMEM_SHARED((RNP,), jnp.float32))
  scratch += [pltpu.SemaphoreType.DMA] * (2 * nbuf + 1)
  out_type = [jax.ShapeDtypeStruct((NC, N, d_feat), jnp.float32)]
  if with_table:
    out_type.append(jax.ShapeDtypeStruct((E,), jnp.float32))

  def body(*args):
    if with_table:
      (hall, ei, et, acc_out, norm_out,
       srcc, dstc, etc_, sidx, normc, *rest) = args
      recip = None
    else:
      (hall, ei, et, recip, acc_out,
       srcc, dstc, etc_, sidx, normc, *rest) = args
      norm_out = None
    rows_bufs = list(rest[:nbuf])
    rest = rest[nbuf:]
    if with_table:
      zb, acc, dbuf, onesb, degacc, *sems = rest
    else:
      zb, acc, *sems = rest
      dbuf = onesb = degacc = None
    sg = list(sems[:nbuf])
    ss = list(sems[nbuf:2 * nbuf])
    sn = sems[2 * nbuf]
    _edge_body(d_feat, with_table, hall, ei, et, recip, acc_out,
               norm_out, srcc, dstc, etc_, sidx, normc,
               rows_bufs, zb, acc, dbuf, onesb, degacc, sg, ss, sn)

  return pl.kernel(body, out_type=out_type, mesh=_mesh,
                   scratch_types=scratch, compiler_params=_sc_params)


_edge1_call = _edge_call(D_HID, True, NBUF1)
_edge2_call = _edge_call(D_OUT, False, NBUF2)


NB = 10
BN = N // NB


def _tc1_body(x_ref, x2_ref, w_ref, root_ref, b_ref, hall_ref, xr_ref):
  x2b = x2_ref[...]
  for r in range(R):
    hall_ref[r] = jnp.dot(x2b, w_ref[r], preferred_element_type=jnp.float32)
  xb = x_ref[...]
  xr_ref[...] = (jnp.dot(xb, root_ref[...], preferred_element_type=jnp.float32)
                 + b_ref[...])


NB1 = 5
BN1 = N // NB1


def _tc1(x, W1, root1, b1):
  wp = jnp.zeros((R, 2 * D_IN, 2 * D_HID), jnp.float32)
  wp = wp.at[:, :D_IN, :D_HID].set(W1).at[:, D_IN:, D_HID:].set(W1)
  return pl.pallas_call(
      _tc1_body,
      grid=(NB1,),
      in_specs=[
          pl.BlockSpec((BN1, D_IN), lambda i: (i, 0)),
          pl.BlockSpec((BN1 // 2, 2 * D_IN), lambda i: (i, 0)),
          pl.BlockSpec((R, 2 * D_IN, 2 * D_HID), lambda i: (0, 0, 0)),
          pl.BlockSpec((D_IN, D_HID), lambda i: (0, 0)),
          pl.BlockSpec((1, D_HID), lambda i: (0, 0)),
      ],
      out_specs=[
          pl.BlockSpec((R, BN1 // 2, 2 * D_HID), lambda i: (0, i, 0)),
          pl.BlockSpec((BN1, D_HID), lambda i: (i, 0)),
      ],
      out_shape=[
          jax.ShapeDtypeStruct((R, N // 2, 2 * D_HID), jnp.float32),
          jax.ShapeDtypeStruct((N, D_HID), jnp.float32),
      ],
  )(x, x.reshape(N // 2, 2 * D_IN), wp, root1, b1)


def _tc2_body(acc_ref, xr_ref, w_ref, root_ref, b_ref, hall_ref, xr2_ref):
  hb = jnp.maximum(acc_ref[0] + acc_ref[1] + xr_ref[...], 0.0)
  for r in range(R):
    hall_ref[r] = jnp.dot(hb, w_ref[r], preferred_element_type=jnp.float32)
  xr2_ref[...] = (jnp.dot(hb, root_ref[...], preferred_element_type=jnp.float32)
                  + b_ref[...])


def _tc2(acc1, xr1, W2, root2, b2):
  return pl.pallas_call(
      _tc2_body,
      grid=(NB,),
      in_specs=[
          pl.BlockSpec((2, BN, D_HID), lambda i: (0, i, 0)),
          pl.BlockSpec((BN, D_HID), lambda i: (i, 0)),
          pl.BlockSpec((R, D_HID, D_OUT), lambda i: (0, 0, 0)),
          pl.BlockSpec((D_HID, D_OUT), lambda i: (0, 0)),
          pl.BlockSpec((1, D_OUT), lambda i: (0, 0)),
      ],
      out_specs=[
          pl.BlockSpec((R, BN, D_OUT), lambda i: (0, i, 0)),
          pl.BlockSpec((BN, D_OUT), lambda i: (i, 0)),
      ],
      out_shape=[
          jax.ShapeDtypeStruct((R, N, D_OUT), jnp.float32),
          jax.ShapeDtypeStruct((N, D_OUT), jnp.float32),
      ],
  )(acc1, xr1, W2, root2, b2)


def _tc3_body(acc_ref, xr_ref, out_ref):
  out_ref[...] = acc_ref[0] + acc_ref[1] + xr_ref[...]


def _tc3(acc2, xr2):
  return pl.pallas_call(
      _tc3_body,
      grid=(NB,),
      in_specs=[
          pl.BlockSpec((2, BN, D_OUT), lambda i: (0, i, 0)),
          pl.BlockSpec((BN, D_OUT), lambda i: (i, 0)),
      ],
      out_specs=pl.BlockSpec((BN, D_OUT), lambda i: (i, 0)),
      out_shape=jax.ShapeDtypeStruct((N, D_OUT), jnp.float32),
  )(acc2, xr2)


@jax.jit
def kernel(x, edge_index, edge_type, W1, root1, b1, W2, root2, b2):
  hall1, xr1 = _tc1(x, W1, root1, b1.reshape(1, D_HID))
  acc1, norm_e = _edge1_call(hall1.reshape(RN, D_HID), edge_index, edge_type)
  hall2, xr2 = _tc2(acc1, xr1, W2, root2, b2.reshape(1, D_OUT))
  (acc2,) = _edge2_call(hall2.reshape(RN, D_OUT), edge_index, edge_type,
                        norm_e)
  return _tc3(acc2, xr2)
